# baseline (speedup 1.0000x reference)
import functools

import jax
import jax.numpy as jnp
from jax import lax
from jax.experimental import pallas as pl
from jax.experimental.pallas import tpu as pltpu
from jax.experimental.pallas import tpu_sc as plsc

ROWS = 4096
COLS = 8192
HALF = COLS // 2
SLABS_PER_CORE = 8
SC_COLS = 2 * SLABS_PER_CORE * 128
TC_COLS = HALF - SC_COLS
CW = 128
NG = CW // 16
HROWS = ROWS // 2
CHUNK = 256
NCHUNK = HROWS // CHUNK
TC_BLK = 512
SC_BLKS = SC_COLS // TC_BLK
BIG = 1 << 30


def _chunk_copy(in_hbm, row_base, col_base, k, buf, sem):
    return pltpu.make_async_copy(
        in_hbm.at[pl.ds(row_base + k * CHUNK, CHUNK), pl.ds(col_base, CW)],
        buf, sem)


def _sc_body(in_hbm, v0_hbm, v1_hbm, i0_hbm, buf_a, buf_b,
             v0_v, v1_v, i0_v, r_v0, r_v1, r_i0,
             sh_v0, sh_v1, sh_i0, sem_a, sem_b):
    core = lax.axis_index("c")
    sub = lax.axis_index("s")
    slab = sub % SLABS_PER_CORE
    upper = sub // SLABS_PER_CORE
    col0 = (core * SLABS_PER_CORE + slab) * CW
    row0 = upper * HROWS

    bufs = (buf_a, buf_b)
    sems = (sem_a, sem_b)

    neg = jnp.full((16,), -jnp.inf, jnp.float32)
    zero_i = jnp.zeros((16,), jnp.int32)

    def top2_row(buf, k, r, c):
        v0s, v1s, i0s = c
        rv = jnp.full((16,), 0, jnp.int32) + (row0 + k * CHUNK + r)
        nv0, nv1, ni0 = [], [], []
        for g in range(NG):
            x = buf[r, pl.ds(g * 16, 16)]
            v0, v1, i0 = v0s[g], v1s[g], i0s[g]
            gt = x > v0
            nv1.append(jnp.maximum(v1, jnp.minimum(x, v0)))
            ni0.append(jnp.where(gt, rv, i0))
            nv0.append(jnp.maximum(v0, x))
        return (tuple(nv0), tuple(nv1), tuple(ni0))

    _chunk_copy(in_hbm, row0, col0, 0, bufs[0], sems[0]).start()

    def outer(t, carry):
        for b in range(2):
            k = t * 2 + b
            _chunk_copy(in_hbm, row0, col0, k, bufs[b], sems[b]).wait()

            @pl.when(k + 1 < NCHUNK)
            def _():
                _chunk_copy(in_hbm, row0, col0, k + 1,
                            bufs[1 - b], sems[1 - b]).start()

            carry = lax.fori_loop(
                0, CHUNK, functools.partial(top2_row, bufs[b], k), carry)
        return carry

    init = (tuple(neg for _ in range(NG)),
            tuple(neg for _ in range(NG)),
            tuple(zero_i for _ in range(NG)))
    v0s, v1s, i0s = lax.fori_loop(0, NCHUNK // 2, outer, init)

    for g in range(NG):
        v0_v[pl.ds(g * 16, 16)] = v0s[g]
        v1_v[pl.ds(g * 16, 16)] = v1s[g]
        i0_v[pl.ds(g * 16, 16)] = i0s[g]

    @pl.when(upper == 1)
    def _():
        pltpu.sync_copy(v0_v, sh_v0.at[slab])
        pltpu.sync_copy(v1_v, sh_v1.at[slab])
        pltpu.sync_copy(i0_v, sh_i0.at[slab])

    plsc.subcore_barrier()

    @pl.when(upper == 0)
    def _():
        pltpu.sync_copy(sh_v0.at[slab], r_v0)
        pltpu.sync_copy(sh_v1.at[slab], r_v1)
        pltpu.sync_copy(sh_i0.at[slab], r_i0)
        for g in range(NG):
            a0, a1, ai = v0s[g], v1s[g], i0s[g]
            b0 = r_v0[pl.ds(g * 16, 16)]
            b1 = r_v1[pl.ds(g * 16, 16)]
            bi = r_i0[pl.ds(g * 16, 16)]
            gt = b0 > a0
            v0_v[pl.ds(g * 16, 16)] = jnp.maximum(a0, b0)
            v1_v[pl.ds(g * 16, 16)] = jnp.maximum(jnp.minimum(a0, b0),
                                                  jnp.maximum(a1, b1))
            i0_v[pl.ds(g * 16, 16)] = jnp.where(gt, bi, ai)
        pltpu.sync_copy(v0_v, v0_hbm.at[pl.ds(col0, CW)])
        pltpu.sync_copy(v1_v, v1_hbm.at[pl.ds(col0, CW)])
        pltpu.sync_copy(i0_v, i0_hbm.at[pl.ds(col0, CW)])


def _sc_top2(inputs):
    mesh = plsc.VectorSubcoreMesh(core_axis_name="c", subcore_axis_name="s")
    shp = jax.ShapeDtypeStruct((SC_COLS,), jnp.float32)
    f = pl.kernel(
        _sc_body,
        out_type=(shp, shp, jax.ShapeDtypeStruct((SC_COLS,), jnp.int32)),
        mesh=mesh,
        scratch_types=[
            pltpu.VMEM((CHUNK, CW), jnp.float32),
            pltpu.VMEM((CHUNK, CW), jnp.float32),
            pltpu.VMEM((CW,), jnp.float32),
            pltpu.VMEM((CW,), jnp.float32),
            pltpu.VMEM((CW,), jnp.int32),
            pltpu.VMEM((CW,), jnp.float32),
            pltpu.VMEM((CW,), jnp.float32),
            pltpu.VMEM((CW,), jnp.int32),
            pltpu.VMEM_SHARED((SLABS_PER_CORE, CW), jnp.float32),
            pltpu.VMEM_SHARED((SLABS_PER_CORE, CW), jnp.float32),
            pltpu.VMEM_SHARED((SLABS_PER_CORE, CW), jnp.int32),
            pltpu.SemaphoreType.DMA,
            pltpu.SemaphoreType.DMA,
        ],
    )
    return f(inputs)


RT_ROWS = 512
RT = ROWS // RT_ROWS


def _tile_stats(x, t):
    tm = jnp.max(x, axis=0)
    rows = lax.broadcasted_iota(jnp.int32, x.shape, 0) + t * RT_ROWS
    eq = x == tm[None, :]
    ti = jnp.min(jnp.where(eq, rows, BIG), axis=0)
    tless = jnp.max(jnp.where(eq, -jnp.inf, x), axis=0)
    cnt = jnp.sum(eq.astype(jnp.float32), axis=0)
    tv1 = jnp.where(cnt > 1.0, tm, tless)
    return tm, tv1, ti


def _tc_full_body(x1_ref, x2_ref, out_ref, v0_s, v1_s, i0_s, m2_s, i1_s):
    t = pl.program_id(1)

    @pl.when(t == 0)
    def _():
        v0_s[...] = jnp.full_like(v0_s[...], -jnp.inf)
        v1_s[...] = jnp.full_like(v1_s[...], -jnp.inf)
        i0_s[...] = jnp.zeros_like(i0_s[...])
        m2_s[...] = jnp.full_like(m2_s[...], -jnp.inf)
        i1_s[...] = jnp.zeros_like(i1_s[...])

    tm, tv1, ti = _tile_stats(x1_ref[...], t)
    v0, v1, i0 = v0_s[...], v1_s[...], i0_s[...]
    gt = tm > v0
    v1_s[...] = jnp.maximum(jnp.minimum(v0, tm), jnp.maximum(v1, tv1))
    i0_s[...] = jnp.where(gt, ti, i0)
    v0_s[...] = jnp.maximum(v0, tm)

    x2 = x2_ref[...]
    m2 = jnp.max(x2, axis=0)[None, :]
    rows2 = lax.broadcasted_iota(jnp.int32, x2.shape, 0) + t * RT_ROWS
    ti2 = jnp.min(jnp.where(x2 == m2, rows2, BIG), axis=0)[None, :]
    pm2, pi1 = m2_s[...], i1_s[...]
    gt2 = m2 > pm2
    i1_s[...] = jnp.where(gt2, ti2, pi1)
    m2_s[...] = jnp.maximum(pm2, m2)

    @pl.when(t == RT - 1)
    def _():
        v0f, v1f = v0_s[...], v1_s[...]
        pen = jnp.where(i0_s[...] == i1_s[...],
                        jnp.float32(-1e16), jnp.float32(0.0))
        out_ref[...] = jnp.maximum(v0f + v0f + pen, v0f + v1f)


def _tc_full(inputs):
    grid = TC_COLS // TC_BLK
    return pl.pallas_call(
        _tc_full_body,
        grid=(grid, RT),
        in_specs=[
            pl.BlockSpec((RT_ROWS, TC_BLK), lambda j, t: (t, SC_BLKS + j)),
            pl.BlockSpec((RT_ROWS, TC_BLK),
                         lambda j, t: (t, HALF // TC_BLK + SC_BLKS + j)),
        ],
        out_specs=pl.BlockSpec((1, TC_BLK), lambda j, t: (0, j)),
        out_shape=jax.ShapeDtypeStruct((1, TC_COLS), jnp.float32),
        scratch_shapes=[pltpu.VMEM((1, TC_BLK), jnp.float32)] * 2
        + [pltpu.VMEM((1, TC_BLK), jnp.int32)]
        + [pltpu.VMEM((1, TC_BLK), jnp.float32),
           pltpu.VMEM((1, TC_BLK), jnp.int32)],
        compiler_params=pltpu.CompilerParams(
            dimension_semantics=("parallel", "arbitrary")),
    )(inputs, inputs)


def _tc_argmax_body(x_ref, i1_ref, m_s, i_s):
    t = pl.program_id(1)

    @pl.when(t == 0)
    def _():
        m_s[...] = jnp.full_like(m_s[...], -jnp.inf)
        i_s[...] = jnp.zeros_like(i_s[...])

    x = x_ref[...]
    tm = jnp.max(x, axis=0)[None, :]
    rows = lax.broadcasted_iota(jnp.int32, x.shape, 0) + t * RT_ROWS
    ti = jnp.min(jnp.where(x == tm, rows, BIG), axis=0)[None, :]
    pm, pi = m_s[...], i_s[...]
    gt = tm > pm
    i_s[...] = jnp.where(gt, ti, pi)
    m_s[...] = jnp.maximum(pm, tm)

    @pl.when(t == RT - 1)
    def _():
        i1_ref[...] = i_s[...]


def _tc_argmax_sc_partners(inputs):
    return pl.pallas_call(
        _tc_argmax_body,
        grid=(SC_BLKS, RT),
        in_specs=[pl.BlockSpec((RT_ROWS, TC_BLK),
                               lambda j, t: (t, HALF // TC_BLK + j))],
        out_specs=pl.BlockSpec((1, TC_BLK), lambda j, t: (0, j)),
        out_shape=jax.ShapeDtypeStruct((1, SC_COLS), jnp.int32),
        scratch_shapes=[pltpu.VMEM((1, TC_BLK), jnp.float32),
                        pltpu.VMEM((1, TC_BLK), jnp.int32)],
        compiler_params=pltpu.CompilerParams(
            dimension_semantics=("parallel", "arbitrary")),
    )(inputs)


def _tc_merge_body(v0_ref, v1_ref, i0_ref, i1_ref, tc_ref, out_ref):
    v0 = v0_ref[...]
    v1 = v1_ref[...]
    pen = jnp.where(i0_ref[...] == i1_ref[...],
                    jnp.float32(-1e16), jnp.float32(0.0))
    out_ref[:, :SC_COLS] = jnp.maximum(v0 + v0 + pen, v0 + v1)
    out_ref[:, SC_COLS:] = tc_ref[...]


def _tc_merge(v0, v1, i0, i1, tc_out):
    return pl.pallas_call(
        _tc_merge_body,
        out_shape=jax.ShapeDtypeStruct((1, HALF), jnp.float32),
    )(v0.reshape(1, SC_COLS), v1.reshape(1, SC_COLS),
      i0.reshape(1, SC_COLS), i1, tc_out)


@jax.jit
def _run(inputs):
    v0, v1, i0 = _sc_top2(inputs)
    tc_out = _tc_full(inputs)
    i1 = _tc_argmax_sc_partners(inputs)
    return _tc_merge(v0, v1, i0, i1, tc_out)


def kernel(inputs):
    return _run(inputs)


# speedup vs baseline: 1.3041x; 1.3041x over previous
import functools

import jax
import jax.numpy as jnp
from jax import lax
from jax.experimental import pallas as pl
from jax.experimental.pallas import tpu as pltpu
from jax.experimental.pallas import tpu_sc as plsc

ROWS = 4096
COLS = 8192
HALF = COLS // 2
SLABS_PER_CORE = 8
SC_COLS = 2 * SLABS_PER_CORE * 128
TC_COLS = HALF - SC_COLS
CW = 128
NG = CW // 16
HROWS = ROWS // 2
CHUNK = 256
NCHUNK = HROWS // CHUNK
TC_BLK = 512
SC_BLKS = SC_COLS // TC_BLK
BIG = 1 << 30


def _chunk_copy(in_hbm, row_base, col_base, k, buf, sem):
    return pltpu.make_async_copy(
        in_hbm.at[pl.ds(row_base + k * CHUNK, CHUNK), pl.ds(col_base, CW)],
        buf, sem)


def _scan(in_hbm, row0, col0, bufs, sems, init, row_body):
    _chunk_copy(in_hbm, row0, col0, 0, bufs[0], sems[0]).start()

    def outer(t, carry):
        for b in range(2):
            k = t * 2 + b
            _chunk_copy(in_hbm, row0, col0, k, bufs[b], sems[b]).wait()

            @pl.when(k + 1 < NCHUNK)
            def _():
                _chunk_copy(in_hbm, row0, col0, k + 1,
                            bufs[1 - b], sems[1 - b]).start()

            carry = lax.fori_loop(
                0, CHUNK, functools.partial(row_body, bufs[b], k), carry)
        return carry

    return lax.fori_loop(0, NCHUNK // 2, outer, init)


def _sc_body(in_hbm, v0_hbm, v1_hbm, i0_hbm, i1_hbm, buf_a, buf_b,
             v0_v, v1_v, i0_v, m2_v, i1_v,
             r_v0, r_v1, r_i0, r_m2, r_i1,
             sh_v0, sh_v1, sh_i0, sh_m2, sh_i1, sem_a, sem_b):
    core = lax.axis_index("c")
    sub = lax.axis_index("s")
    slab = sub % SLABS_PER_CORE
    upper = sub // SLABS_PER_CORE
    col0 = (core * SLABS_PER_CORE + slab) * CW
    row0 = upper * HROWS

    bufs = (buf_a, buf_b)
    sems = (sem_a, sem_b)

    neg = jnp.full((16,), -jnp.inf, jnp.float32)
    zero_i = jnp.zeros((16,), jnp.int32)

    def top2_row(buf, k, r, c):
        v0s, v1s, i0s = c
        rv = jnp.full((16,), 0, jnp.int32) + (row0 + k * CHUNK + r)
        nv0, nv1, ni0 = [], [], []
        for g in range(NG):
            x = buf[r, pl.ds(g * 16, 16)]
            v0, v1, i0 = v0s[g], v1s[g], i0s[g]
            gt = x > v0
            nv1.append(jnp.maximum(v1, jnp.minimum(x, v0)))
            ni0.append(jnp.where(gt, rv, i0))
            nv0.append(jnp.maximum(v0, x))
        return (tuple(nv0), tuple(nv1), tuple(ni0))

    def argmax_row(buf, k, r, c):
        ms, i1s = c
        rv = jnp.full((16,), 0, jnp.int32) + (row0 + k * CHUNK + r)
        nm, ni1 = [], []
        for g in range(NG):
            x = buf[r, pl.ds(g * 16, 16)]
            m, i1 = ms[g], i1s[g]
            gt = x > m
            ni1.append(jnp.where(gt, rv, i1))
            nm.append(jnp.maximum(m, x))
        return (tuple(nm), tuple(ni1))

    init1 = (tuple(neg for _ in range(NG)),
             tuple(neg for _ in range(NG)),
             tuple(zero_i for _ in range(NG)))
    v0s, v1s, i0s = _scan(in_hbm, row0, col0, bufs, sems, init1, top2_row)

    init2 = (tuple(neg for _ in range(NG)),
             tuple(zero_i for _ in range(NG)))
    m2s, i1s = _scan(in_hbm, row0, HALF + col0, bufs, sems, init2, argmax_row)

    for g in range(NG):
        sl = pl.ds(g * 16, 16)
        v0_v[sl] = v0s[g]
        v1_v[sl] = v1s[g]
        i0_v[sl] = i0s[g]
        m2_v[sl] = m2s[g]
        i1_v[sl] = i1s[g]

    @pl.when(upper == 1)
    def _():
        pltpu.sync_copy(v0_v, sh_v0.at[slab])
        pltpu.sync_copy(v1_v, sh_v1.at[slab])
        pltpu.sync_copy(i0_v, sh_i0.at[slab])
        pltpu.sync_copy(m2_v, sh_m2.at[slab])
        pltpu.sync_copy(i1_v, sh_i1.at[slab])

    plsc.subcore_barrier()

    @pl.when(upper == 0)
    def _():
        pltpu.sync_copy(sh_v0.at[slab], r_v0)
        pltpu.sync_copy(sh_v1.at[slab], r_v1)
        pltpu.sync_copy(sh_i0.at[slab], r_i0)
        pltpu.sync_copy(sh_m2.at[slab], r_m2)
        pltpu.sync_copy(sh_i1.at[slab], r_i1)
        for g in range(NG):
            sl = pl.ds(g * 16, 16)
            a0, a1, ai = v0s[g], v1s[g], i0s[g]
            b0, b1, bi = r_v0[sl], r_v1[sl], r_i0[sl]
            gt = b0 > a0
            v0_v[sl] = jnp.maximum(a0, b0)
            v1_v[sl] = jnp.maximum(jnp.minimum(a0, b0),
                                   jnp.maximum(a1, b1))
            i0_v[sl] = jnp.where(gt, bi, ai)
            am, aj = m2s[g], i1s[g]
            bm, bj = r_m2[sl], r_i1[sl]
            i1_v[sl] = jnp.where(bm > am, bj, aj)
        pltpu.sync_copy(v0_v, v0_hbm.at[pl.ds(col0, CW)])
        pltpu.sync_copy(v1_v, v1_hbm.at[pl.ds(col0, CW)])
        pltpu.sync_copy(i0_v, i0_hbm.at[pl.ds(col0, CW)])
        pltpu.sync_copy(i1_v, i1_hbm.at[pl.ds(col0, CW)])


def _sc_top2(inputs):
    mesh = plsc.VectorSubcoreMesh(core_axis_name="c", subcore_axis_name="s")
    f32 = jnp.float32
    i32 = jnp.int32
    f = pl.kernel(
        _sc_body,
        out_type=(jax.ShapeDtypeStruct((SC_COLS,), f32),
                  jax.ShapeDtypeStruct((SC_COLS,), f32),
                  jax.ShapeDtypeStruct((SC_COLS,), i32),
                  jax.ShapeDtypeStruct((SC_COLS,), i32)),
        mesh=mesh,
        scratch_types=[
            pltpu.VMEM((CHUNK, CW), f32),
            pltpu.VMEM((CHUNK, CW), f32),
            pltpu.VMEM((CW,), f32),
            pltpu.VMEM((CW,), f32),
            pltpu.VMEM((CW,), i32),
            pltpu.VMEM((CW,), f32),
            pltpu.VMEM((CW,), i32),
            pltpu.VMEM((CW,), f32),
            pltpu.VMEM((CW,), f32),
            pltpu.VMEM((CW,), i32),
            pltpu.VMEM((CW,), f32),
            pltpu.VMEM((CW,), i32),
            pltpu.VMEM_SHARED((SLABS_PER_CORE, CW), f32),
            pltpu.VMEM_SHARED((SLABS_PER_CORE, CW), f32),
            pltpu.VMEM_SHARED((SLABS_PER_CORE, CW), i32),
            pltpu.VMEM_SHARED((SLABS_PER_CORE, CW), f32),
            pltpu.VMEM_SHARED((SLABS_PER_CORE, CW), i32),
            pltpu.SemaphoreType.DMA,
            pltpu.SemaphoreType.DMA,
        ],
    )
    return f(inputs)


def _colmax_argmax(x):
    m = jnp.max(x, axis=0)
    rows = lax.broadcasted_iota(jnp.int32, x.shape, 0)
    i = jnp.min(jnp.where(x == m[None, :], rows, BIG), axis=0)
    return m, i, rows


def _tc_full_body(x1_ref, x2_ref, out_ref):
    x1 = x1_ref[...]
    v0, i0, rows = _colmax_argmax(x1)
    v1 = jnp.max(jnp.where(rows == i0[None, :], -jnp.inf, x1), axis=0)
    x2 = x2_ref[...]
    _, i1, _ = _colmax_argmax(x2)
    pen = jnp.where(i0 == i1, jnp.float32(-1e16), jnp.float32(0.0))
    out_ref[...] = jnp.maximum(v0 + v0 + pen, v0 + v1)[None, :]


def _tc_full(inputs):
    grid = TC_COLS // TC_BLK
    return pl.pallas_call(
        _tc_full_body,
        grid=(grid,),
        in_specs=[
            pl.BlockSpec((ROWS, TC_BLK), lambda j: (0, SC_BLKS + j)),
            pl.BlockSpec((ROWS, TC_BLK),
                         lambda j: (0, HALF // TC_BLK + SC_BLKS + j)),
        ],
        out_specs=pl.BlockSpec((1, TC_BLK), lambda j: (0, j)),
        out_shape=jax.ShapeDtypeStruct((1, TC_COLS), jnp.float32),
    )(inputs, inputs)


def _tc_merge_body(v0_ref, v1_ref, i0_ref, i1_ref, tc_ref, out_ref):
    v0 = v0_ref[...]
    v1 = v1_ref[...]
    pen = jnp.where(i0_ref[...] == i1_ref[...],
                    jnp.float32(-1e16), jnp.float32(0.0))
    out_ref[:, :SC_COLS] = jnp.maximum(v0 + v0 + pen, v0 + v1)
    out_ref[:, SC_COLS:] = tc_ref[...]


def _tc_merge(v0, v1, i0, i1, tc_out):
    return pl.pallas_call(
        _tc_merge_body,
        out_shape=jax.ShapeDtypeStruct((1, HALF), jnp.float32),
    )(v0.reshape(1, SC_COLS), v1.reshape(1, SC_COLS),
      i0.reshape(1, SC_COLS), i1.reshape(1, SC_COLS), tc_out)


@jax.jit
def _run(inputs):
    v0, v1, i0, i1 = _sc_top2(inputs)
    tc_out = _tc_full(inputs)
    return _tc_merge(v0, v1, i0, i1, tc_out)


def kernel(inputs):
    return _run(inputs)


# speedup vs baseline: 1.3458x; 1.0319x over previous
import functools

import jax
import jax.numpy as jnp
from jax import lax
from jax.experimental import pallas as pl
from jax.experimental.pallas import tpu as pltpu
from jax.experimental.pallas import tpu_sc as plsc

ROWS = 4096
COLS = 8192
HALF = COLS // 2
SLABS_PER_CORE = 8
SC_COLS = 2 * SLABS_PER_CORE * 128
TC_COLS = HALF - SC_COLS
CW = 128
NG = CW // 16
HROWS = ROWS // 2
CHUNK = 256
NCHUNK = HROWS // CHUNK
TC_BLK = 512
SC_BLKS = SC_COLS // TC_BLK
BIG = 1 << 30


def _chunk_copy(in_hbm, row_base, col_base, k, buf, sem):
    return pltpu.make_async_copy(
        in_hbm.at[pl.ds(row_base + k * CHUNK, CHUNK), pl.ds(col_base, CW)],
        buf, sem)


def _scan(in_hbm, row0, col0, bufs, sems, init, row_body,
          prefetch_col=None, skip_prologue=False):
    if not skip_prologue:
        _chunk_copy(in_hbm, row0, col0, 0, bufs[0], sems[0]).start()

    def outer(t, carry):
        for b in range(2):
            k = t * 2 + b
            _chunk_copy(in_hbm, row0, col0, k, bufs[b], sems[b]).wait()

            @pl.when(k + 1 < NCHUNK)
            def _():
                _chunk_copy(in_hbm, row0, col0, k + 1,
                            bufs[1 - b], sems[1 - b]).start()

            if prefetch_col is not None:

                @pl.when(k + 1 == NCHUNK)
                def _():
                    _chunk_copy(in_hbm, row0, prefetch_col, 0,
                                bufs[1 - b], sems[1 - b]).start()

            carry = lax.fori_loop(
                0, CHUNK, functools.partial(row_body, bufs[b], k), carry)
        return carry

    return lax.fori_loop(0, NCHUNK // 2, outer, init)


def _sc_body(in_hbm, out_hbm, buf_a, buf_b,
             v0_v, v1_v, i0_v, m2_v, i1_v,
             r_v0, r_v1, r_i0, r_m2, r_i1,
             sh_v0, sh_v1, sh_i0, sh_m2, sh_i1, sem_a, sem_b):
    core = lax.axis_index("c")
    sub = lax.axis_index("s")
    slab = sub % SLABS_PER_CORE
    upper = sub // SLABS_PER_CORE
    col0 = (core * SLABS_PER_CORE + slab) * CW
    row0 = upper * HROWS

    bufs = (buf_a, buf_b)
    sems = (sem_a, sem_b)

    neg = jnp.full((16,), -jnp.inf, jnp.float32)
    zero_i = jnp.zeros((16,), jnp.int32)

    def top2_row(buf, k, r, c):
        v0s, v1s, i0s = c
        rv = jnp.full((16,), 0, jnp.int32) + (row0 + k * CHUNK + r)
        nv0, nv1, ni0 = [], [], []
        for g in range(NG):
            x = buf[r, pl.ds(g * 16, 16)]
            v0, v1, i0 = v0s[g], v1s[g], i0s[g]
            gt = x > v0
            nv1.append(jnp.maximum(v1, jnp.minimum(x, v0)))
            ni0.append(jnp.where(gt, rv, i0))
            nv0.append(jnp.maximum(v0, x))
        return (tuple(nv0), tuple(nv1), tuple(ni0))

    def argmax_row(buf, k, r, c):
        ms, i1s = c
        rv = jnp.full((16,), 0, jnp.int32) + (row0 + k * CHUNK + r)
        nm, ni1 = [], []
        for g in range(NG):
            x = buf[r, pl.ds(g * 16, 16)]
            m, i1 = ms[g], i1s[g]
            gt = x > m
            ni1.append(jnp.where(gt, rv, i1))
            nm.append(jnp.maximum(m, x))
        return (tuple(nm), tuple(ni1))

    init1 = (tuple(neg for _ in range(NG)),
             tuple(neg for _ in range(NG)),
             tuple(zero_i for _ in range(NG)))
    v0s, v1s, i0s = _scan(in_hbm, row0, col0, bufs, sems, init1, top2_row,
                          prefetch_col=HALF + col0)

    init2 = (tuple(neg for _ in range(NG)),
             tuple(zero_i for _ in range(NG)))
    m2s, i1s = _scan(in_hbm, row0, HALF + col0, bufs, sems, init2,
                     argmax_row, skip_prologue=True)

    for g in range(NG):
        sl = pl.ds(g * 16, 16)
        v0_v[sl] = v0s[g]
        v1_v[sl] = v1s[g]
        i0_v[sl] = i0s[g]
        m2_v[sl] = m2s[g]
        i1_v[sl] = i1s[g]

    @pl.when(upper == 1)
    def _():
        pltpu.sync_copy(v0_v, sh_v0.at[slab])
        pltpu.sync_copy(v1_v, sh_v1.at[slab])
        pltpu.sync_copy(i0_v, sh_i0.at[slab])
        pltpu.sync_copy(m2_v, sh_m2.at[slab])
        pltpu.sync_copy(i1_v, sh_i1.at[slab])

    plsc.subcore_barrier()

    @pl.when(upper == 0)
    def _():
        pltpu.sync_copy(sh_v0.at[slab], r_v0)
        pltpu.sync_copy(sh_v1.at[slab], r_v1)
        pltpu.sync_copy(sh_i0.at[slab], r_i0)
        pltpu.sync_copy(sh_m2.at[slab], r_m2)
        pltpu.sync_copy(sh_i1.at[slab], r_i1)
        pen_v = jnp.full((16,), -1e16, jnp.float32)
        zero_f = jnp.zeros((16,), jnp.float32)
        for g in range(NG):
            sl = pl.ds(g * 16, 16)
            a0, a1, ai = v0s[g], v1s[g], i0s[g]
            b0, b1, bi = r_v0[sl], r_v1[sl], r_i0[sl]
            gt = b0 > a0
            v0 = jnp.maximum(a0, b0)
            v1 = jnp.maximum(jnp.minimum(a0, b0), jnp.maximum(a1, b1))
            i0 = jnp.where(gt, bi, ai)
            am, aj = m2s[g], i1s[g]
            bm, bj = r_m2[sl], r_i1[sl]
            i1 = jnp.where(bm > am, bj, aj)
            pen = jnp.where(i0 == i1, pen_v, zero_f)
            v0_v[sl] = jnp.maximum(v0 + v0 + pen, v0 + v1)
        pltpu.sync_copy(v0_v, out_hbm.at[pl.ds(col0, CW)])


def _sc_part(inputs):
    mesh = plsc.VectorSubcoreMesh(core_axis_name="c", subcore_axis_name="s")
    f32 = jnp.float32
    i32 = jnp.int32
    f = pl.kernel(
        _sc_body,
        out_type=jax.ShapeDtypeStruct((SC_COLS,), f32),
        mesh=mesh,
        scratch_types=[
            pltpu.VMEM((CHUNK, CW), f32),
            pltpu.VMEM((CHUNK, CW), f32),
            pltpu.VMEM((CW,), f32),
            pltpu.VMEM((CW,), f32),
            pltpu.VMEM((CW,), i32),
            pltpu.VMEM((CW,), f32),
            pltpu.VMEM((CW,), i32),
            pltpu.VMEM((CW,), f32),
            pltpu.VMEM((CW,), f32),
            pltpu.VMEM((CW,), i32),
            pltpu.VMEM((CW,), f32),
            pltpu.VMEM((CW,), i32),
            pltpu.VMEM_SHARED((SLABS_PER_CORE, CW), f32),
            pltpu.VMEM_SHARED((SLABS_PER_CORE, CW), f32),
            pltpu.VMEM_SHARED((SLABS_PER_CORE, CW), i32),
            pltpu.VMEM_SHARED((SLABS_PER_CORE, CW), f32),
            pltpu.VMEM_SHARED((SLABS_PER_CORE, CW), i32),
            pltpu.SemaphoreType.DMA,
            pltpu.SemaphoreType.DMA,
        ],
    )
    return f(inputs)


def _colmax_argmax(x):
    m = jnp.max(x, axis=0)
    rows = lax.broadcasted_iota(jnp.int32, x.shape, 0)
    i = jnp.min(jnp.where(x == m[None, :], rows, BIG), axis=0)
    return m, i, rows


def _tc_full_body(x1_ref, x2_ref, out_ref):
    x1 = x1_ref[...]
    v0, i0, rows = _colmax_argmax(x1)
    v1 = jnp.max(jnp.where(rows == i0[None, :], -jnp.inf, x1), axis=0)
    x2 = x2_ref[...]
    _, i1, _ = _colmax_argmax(x2)
    pen = jnp.where(i0 == i1, jnp.float32(-1e16), jnp.float32(0.0))
    out_ref[...] = jnp.maximum(v0 + v0 + pen, v0 + v1)[None, :]


def _tc_full(inputs):
    grid = TC_COLS // TC_BLK
    return pl.pallas_call(
        _tc_full_body,
        grid=(grid,),
        in_specs=[
            pl.BlockSpec((ROWS, TC_BLK), lambda j: (0, SC_BLKS + j)),
            pl.BlockSpec((ROWS, TC_BLK),
                         lambda j: (0, HALF // TC_BLK + SC_BLKS + j)),
        ],
        out_specs=pl.BlockSpec((1, TC_BLK), lambda j: (0, j)),
        out_shape=jax.ShapeDtypeStruct((1, TC_COLS), jnp.float32),
    )(inputs, inputs)


@jax.jit
def _run(inputs):
    out_sc = _sc_part(inputs)
    out_tc = _tc_full(inputs)
    return jnp.concatenate([out_sc.reshape(1, SC_COLS), out_tc], axis=1)


def kernel(inputs):
    return _run(inputs)


# speedup vs baseline: 1.3896x; 1.0326x over previous
import functools

import jax
import jax.numpy as jnp
from jax import lax
from jax.experimental import pallas as pl
from jax.experimental.pallas import tpu as pltpu
from jax.experimental.pallas import tpu_sc as plsc

ROWS = 4096
COLS = 8192
HALF = COLS // 2
SLABS_PER_CORE = 8
SC_COLS = 2 * SLABS_PER_CORE * 128
TC_COLS = HALF - SC_COLS
CW = 128
NG = CW // 16
HROWS = ROWS // 2
CHUNK = 256
NCHUNK = HROWS // CHUNK
TC_BLK = 512
SC_BLKS = SC_COLS // TC_BLK
BIG = 1 << 30


def _chunk_copy(in_hbm, row_base, col_base, k, buf, sem):
    return pltpu.make_async_copy(
        in_hbm.at[pl.ds(row_base + k * CHUNK, CHUNK), pl.ds(col_base, CW)],
        buf, sem)


NBUF = 3


def _sc_body(in_hbm, out_hbm, buf_a, buf_b, buf_c,
             v0_v, v1_v, i0_v, m2_v, i1_v,
             r_v0, r_v1, r_i0, r_m2, r_i1,
             sh_v0, sh_v1, sh_i0, sh_m2, sh_i1, sem_a, sem_b, sem_c):
    core = lax.axis_index("c")
    sub = lax.axis_index("s")
    slab = sub % SLABS_PER_CORE
    upper = sub // SLABS_PER_CORE
    col0 = (core * SLABS_PER_CORE + slab) * CW
    row0 = upper * HROWS

    bufs = (buf_a, buf_b, buf_c)
    sems = (sem_a, sem_b, sem_c)

    neg = jnp.full((16,), -jnp.inf, jnp.float32)
    zero_i = jnp.zeros((16,), jnp.int32)

    def top2_row(buf, k, r, c):
        v0s, v1s, i0s = c
        rv = jnp.full((16,), 0, jnp.int32) + (row0 + k * CHUNK + r)
        nv0, nv1, ni0 = [], [], []
        for g in range(NG):
            x = buf[r, pl.ds(g * 16, 16)]
            v0, v1, i0 = v0s[g], v1s[g], i0s[g]
            gt = x > v0
            nv1.append(jnp.maximum(v1, jnp.minimum(x, v0)))
            ni0.append(jnp.where(gt, rv, i0))
            nv0.append(jnp.maximum(v0, x))
        return (tuple(nv0), tuple(nv1), tuple(ni0))

    def argmax_row(buf, k, r, c):
        ms, i1s = c
        rv = jnp.full((16,), 0, jnp.int32) + (row0 + k * CHUNK + r)
        nm, ni1 = [], []
        for g in range(NG):
            x = buf[r, pl.ds(g * 16, 16)]
            m, i1 = ms[g], i1s[g]
            gt = x > m
            ni1.append(jnp.where(gt, rv, i1))
            nm.append(jnp.maximum(m, x))
        return (tuple(nm), tuple(ni1))

    chunks = ([(col0, k) for k in range(NCHUNK)]
              + [(HALF + col0, k) for k in range(NCHUNK)])
    nch = len(chunks)

    def copy_j(j):
        cb, k = chunks[j]
        s = j % NBUF
        return _chunk_copy(in_hbm, row0, cb, k, bufs[s], sems[s])

    copy_j(0).start()
    copy_j(1).start()

    carry1 = (tuple(neg for _ in range(NG)),
              tuple(neg for _ in range(NG)),
              tuple(zero_i for _ in range(NG)))
    carry2 = (tuple(neg for _ in range(NG)),
              tuple(zero_i for _ in range(NG)))
    for j in range(nch):
        copy_j(j).wait()
        if j + 2 < nch:
            copy_j(j + 2).start()
        buf, k = bufs[j % NBUF], chunks[j][1]
        if j < NCHUNK:
            carry1 = lax.fori_loop(
                0, CHUNK, functools.partial(top2_row, buf, k), carry1)
        else:
            carry2 = lax.fori_loop(
                0, CHUNK, functools.partial(argmax_row, buf, k), carry2)
    v0s, v1s, i0s = carry1
    m2s, i1s = carry2

    for g in range(NG):
        sl = pl.ds(g * 16, 16)
        v0_v[sl] = v0s[g]
        v1_v[sl] = v1s[g]
        i0_v[sl] = i0s[g]
        m2_v[sl] = m2s[g]
        i1_v[sl] = i1s[g]

    @pl.when(upper == 1)
    def _():
        pltpu.sync_copy(v0_v, sh_v0.at[slab])
        pltpu.sync_copy(v1_v, sh_v1.at[slab])
        pltpu.sync_copy(i0_v, sh_i0.at[slab])
        pltpu.sync_copy(m2_v, sh_m2.at[slab])
        pltpu.sync_copy(i1_v, sh_i1.at[slab])

    plsc.subcore_barrier()

    @pl.when(upper == 0)
    def _():
        pltpu.sync_copy(sh_v0.at[slab], r_v0)
        pltpu.sync_copy(sh_v1.at[slab], r_v1)
        pltpu.sync_copy(sh_i0.at[slab], r_i0)
        pltpu.sync_copy(sh_m2.at[slab], r_m2)
        pltpu.sync_copy(sh_i1.at[slab], r_i1)
        pen_v = jnp.full((16,), -1e16, jnp.float32)
        zero_f = jnp.zeros((16,), jnp.float32)
        for g in range(NG):
            sl = pl.ds(g * 16, 16)
            a0, a1, ai = v0s[g], v1s[g], i0s[g]
            b0, b1, bi = r_v0[sl], r_v1[sl], r_i0[sl]
            gt = b0 > a0
            v0 = jnp.maximum(a0, b0)
            v1 = jnp.maximum(jnp.minimum(a0, b0), jnp.maximum(a1, b1))
            i0 = jnp.where(gt, bi, ai)
            am, aj = m2s[g], i1s[g]
            bm, bj = r_m2[sl], r_i1[sl]
            i1 = jnp.where(bm > am, bj, aj)
            pen = jnp.where(i0 == i1, pen_v, zero_f)
            v0_v[sl] = jnp.maximum(v0 + v0 + pen, v0 + v1)
        pltpu.sync_copy(v0_v, out_hbm.at[pl.ds(col0, CW)])


def _sc_part(inputs):
    mesh = plsc.VectorSubcoreMesh(core_axis_name="c", subcore_axis_name="s")
    f32 = jnp.float32
    i32 = jnp.int32
    f = pl.kernel(
        _sc_body,
        out_type=jax.ShapeDtypeStruct((SC_COLS,), f32),
        mesh=mesh,
        scratch_types=[
            pltpu.VMEM((CHUNK, CW), f32),
            pltpu.VMEM((CHUNK, CW), f32),
            pltpu.VMEM((CHUNK, CW), f32),
            pltpu.VMEM((CW,), f32),
            pltpu.VMEM((CW,), f32),
            pltpu.VMEM((CW,), i32),
            pltpu.VMEM((CW,), f32),
            pltpu.VMEM((CW,), i32),
            pltpu.VMEM((CW,), f32),
            pltpu.VMEM((CW,), f32),
            pltpu.VMEM((CW,), i32),
            pltpu.VMEM((CW,), f32),
            pltpu.VMEM((CW,), i32),
            pltpu.VMEM_SHARED((SLABS_PER_CORE, CW), f32),
            pltpu.VMEM_SHARED((SLABS_PER_CORE, CW), f32),
            pltpu.VMEM_SHARED((SLABS_PER_CORE, CW), i32),
            pltpu.VMEM_SHARED((SLABS_PER_CORE, CW), f32),
            pltpu.VMEM_SHARED((SLABS_PER_CORE, CW), i32),
            pltpu.SemaphoreType.DMA,
            pltpu.SemaphoreType.DMA,
            pltpu.SemaphoreType.DMA,
        ],
    )
    return f(inputs)


def _colmax_argmax(x):
    m = jnp.max(x, axis=0)
    rows = lax.broadcasted_iota(jnp.int32, x.shape, 0)
    i = jnp.min(jnp.where(x == m[None, :], rows, BIG), axis=0)
    return m, i, rows


def _tc_full_body(x1_ref, x2_ref, out_ref):
    x1 = x1_ref[...]
    v0, i0, rows = _colmax_argmax(x1)
    v1 = jnp.max(jnp.where(rows == i0[None, :], -jnp.inf, x1), axis=0)
    x2 = x2_ref[...]
    _, i1, _ = _colmax_argmax(x2)
    pen = jnp.where(i0 == i1, jnp.float32(-1e16), jnp.float32(0.0))
    out_ref[...] = jnp.maximum(v0 + v0 + pen, v0 + v1)[None, :]


def _tc_full(inputs):
    grid = TC_COLS // TC_BLK
    return pl.pallas_call(
        _tc_full_body,
        grid=(grid,),
        in_specs=[
            pl.BlockSpec((ROWS, TC_BLK), lambda j: (0, SC_BLKS + j)),
            pl.BlockSpec((ROWS, TC_BLK),
                         lambda j: (0, HALF // TC_BLK + SC_BLKS + j)),
        ],
        out_specs=pl.BlockSpec((1, TC_BLK), lambda j: (0, j)),
        out_shape=jax.ShapeDtypeStruct((1, TC_COLS), jnp.float32),
    )(inputs, inputs)


@jax.jit
def _run(inputs):
    out_sc = _sc_part(inputs)
    out_tc = _tc_full(inputs)
    return jnp.concatenate([out_sc.reshape(1, SC_COLS), out_tc], axis=1)


def kernel(inputs):
    return _run(inputs)


# speedup vs baseline: 1.4117x; 1.0159x over previous
import functools

import jax
import jax.numpy as jnp
from jax import lax
from jax.experimental import pallas as pl
from jax.experimental.pallas import tpu as pltpu
from jax.experimental.pallas import tpu_sc as plsc

ROWS = 4096
COLS = 8192
HALF = COLS // 2
SLABS_PER_CORE = 8
SC_COLS = 2 * SLABS_PER_CORE * 128
TC_COLS = HALF - SC_COLS
CW = 128
NG = CW // 16
HROWS = ROWS // 2
CHUNK = 128
NCHUNK = HROWS // CHUNK
TC_BLK = 512
SC_BLKS = SC_COLS // TC_BLK
BIG = 1 << 30


def _chunk_copy(in_hbm, row_base, col_base, k, buf, sem):
    return pltpu.make_async_copy(
        in_hbm.at[pl.ds(row_base + k * CHUNK, CHUNK), pl.ds(col_base, CW)],
        buf, sem)


NBUF = 4


def _sc_body(in_hbm, out_hbm, buf_a, buf_b, buf_c, buf_d,
             v0_v, v1_v, i0_v, m2_v, i1_v,
             r_v0, r_v1, r_i0, r_m2, r_i1,
             sh_v0, sh_v1, sh_i0, sh_m2, sh_i1,
             sem_a, sem_b, sem_c, sem_d):
    core = lax.axis_index("c")
    sub = lax.axis_index("s")
    slab = sub % SLABS_PER_CORE
    upper = sub // SLABS_PER_CORE
    col0 = (core * SLABS_PER_CORE + slab) * CW
    row0 = upper * HROWS

    bufs = (buf_a, buf_b, buf_c, buf_d)
    sems = (sem_a, sem_b, sem_c, sem_d)

    neg = jnp.full((16,), -jnp.inf, jnp.float32)
    zero_i = jnp.zeros((16,), jnp.int32)

    def top2_row(buf, k, r, c):
        v0s, v1s, i0s = c
        rv = jnp.full((16,), 0, jnp.int32) + (row0 + k * CHUNK + r)
        nv0, nv1, ni0 = [], [], []
        for g in range(NG):
            x = buf[r, pl.ds(g * 16, 16)]
            v0, v1, i0 = v0s[g], v1s[g], i0s[g]
            gt = x > v0
            nv1.append(jnp.maximum(v1, jnp.minimum(x, v0)))
            ni0.append(jnp.where(gt, rv, i0))
            nv0.append(jnp.maximum(v0, x))
        return (tuple(nv0), tuple(nv1), tuple(ni0))

    def argmax_row(buf, k, r, c):
        ms, i1s = c
        rv = jnp.full((16,), 0, jnp.int32) + (row0 + k * CHUNK + r)
        nm, ni1 = [], []
        for g in range(NG):
            x = buf[r, pl.ds(g * 16, 16)]
            m, i1 = ms[g], i1s[g]
            gt = x > m
            ni1.append(jnp.where(gt, rv, i1))
            nm.append(jnp.maximum(m, x))
        return (tuple(nm), tuple(ni1))

    chunks = ([(col0, k) for k in range(NCHUNK)]
              + [(HALF + col0, k) for k in range(NCHUNK)])
    nch = len(chunks)

    def copy_j(j):
        cb, k = chunks[j]
        s = j % NBUF
        return _chunk_copy(in_hbm, row0, cb, k, bufs[s], sems[s])

    copy_j(0).start()
    copy_j(1).start()
    copy_j(2).start()

    carry1 = (tuple(neg for _ in range(NG)),
              tuple(neg for _ in range(NG)),
              tuple(zero_i for _ in range(NG)))
    carry2 = (tuple(neg for _ in range(NG)),
              tuple(zero_i for _ in range(NG)))
    for j in range(nch):
        copy_j(j).wait()
        if j + 3 < nch:
            copy_j(j + 3).start()
        buf, k = bufs[j % NBUF], chunks[j][1]
        if j < NCHUNK:
            carry1 = lax.fori_loop(
                0, CHUNK, functools.partial(top2_row, buf, k), carry1)
        else:
            carry2 = lax.fori_loop(
                0, CHUNK, functools.partial(argmax_row, buf, k), carry2)
    v0s, v1s, i0s = carry1
    m2s, i1s = carry2

    for g in range(NG):
        sl = pl.ds(g * 16, 16)
        v0_v[sl] = v0s[g]
        v1_v[sl] = v1s[g]
        i0_v[sl] = i0s[g]
        m2_v[sl] = m2s[g]
        i1_v[sl] = i1s[g]

    @pl.when(upper == 1)
    def _():
        pltpu.sync_copy(v0_v, sh_v0.at[slab])
        pltpu.sync_copy(v1_v, sh_v1.at[slab])
        pltpu.sync_copy(i0_v, sh_i0.at[slab])
        pltpu.sync_copy(m2_v, sh_m2.at[slab])
        pltpu.sync_copy(i1_v, sh_i1.at[slab])

    plsc.subcore_barrier()

    @pl.when(upper == 0)
    def _():
        pltpu.sync_copy(sh_v0.at[slab], r_v0)
        pltpu.sync_copy(sh_v1.at[slab], r_v1)
        pltpu.sync_copy(sh_i0.at[slab], r_i0)
        pltpu.sync_copy(sh_m2.at[slab], r_m2)
        pltpu.sync_copy(sh_i1.at[slab], r_i1)
        pen_v = jnp.full((16,), -1e16, jnp.float32)
        zero_f = jnp.zeros((16,), jnp.float32)
        for g in range(NG):
            sl = pl.ds(g * 16, 16)
            a0, a1, ai = v0s[g], v1s[g], i0s[g]
            b0, b1, bi = r_v0[sl], r_v1[sl], r_i0[sl]
            gt = b0 > a0
            v0 = jnp.maximum(a0, b0)
            v1 = jnp.maximum(jnp.minimum(a0, b0), jnp.maximum(a1, b1))
            i0 = jnp.where(gt, bi, ai)
            am, aj = m2s[g], i1s[g]
            bm, bj = r_m2[sl], r_i1[sl]
            i1 = jnp.where(bm > am, bj, aj)
            pen = jnp.where(i0 == i1, pen_v, zero_f)
            v0_v[sl] = jnp.maximum(v0 + v0 + pen, v0 + v1)
        pltpu.sync_copy(v0_v, out_hbm.at[pl.ds(col0, CW)])


def _sc_part(inputs):
    mesh = plsc.VectorSubcoreMesh(core_axis_name="c", subcore_axis_name="s")
    f32 = jnp.float32
    i32 = jnp.int32
    f = pl.kernel(
        _sc_body,
        out_type=jax.ShapeDtypeStruct((SC_COLS,), f32),
        mesh=mesh,
        scratch_types=[
            pltpu.VMEM((CHUNK, CW), f32),
            pltpu.VMEM((CHUNK, CW), f32),
            pltpu.VMEM((CHUNK, CW), f32),
            pltpu.VMEM((CHUNK, CW), f32),
            pltpu.VMEM((CW,), f32),
            pltpu.VMEM((CW,), f32),
            pltpu.VMEM((CW,), i32),
            pltpu.VMEM((CW,), f32),
            pltpu.VMEM((CW,), i32),
            pltpu.VMEM((CW,), f32),
            pltpu.VMEM((CW,), f32),
            pltpu.VMEM((CW,), i32),
            pltpu.VMEM((CW,), f32),
            pltpu.VMEM((CW,), i32),
            pltpu.VMEM_SHARED((SLABS_PER_CORE, CW), f32),
            pltpu.VMEM_SHARED((SLABS_PER_CORE, CW), f32),
            pltpu.VMEM_SHARED((SLABS_PER_CORE, CW), i32),
            pltpu.VMEM_SHARED((SLABS_PER_CORE, CW), f32),
            pltpu.VMEM_SHARED((SLABS_PER_CORE, CW), i32),
            pltpu.SemaphoreType.DMA,
            pltpu.SemaphoreType.DMA,
            pltpu.SemaphoreType.DMA,
            pltpu.SemaphoreType.DMA,
        ],
    )
    return f(inputs)


def _colmax_argmax(x):
    m = jnp.max(x, axis=0)
    rows = lax.broadcasted_iota(jnp.int32, x.shape, 0)
    i = jnp.min(jnp.where(x == m[None, :], rows, BIG), axis=0)
    return m, i, rows


def _tc_full_body(x1_ref, x2_ref, out_ref):
    x1 = x1_ref[...]
    v0, i0, rows = _colmax_argmax(x1)
    v1 = jnp.max(jnp.where(rows == i0[None, :], -jnp.inf, x1), axis=0)
    x2 = x2_ref[...]
    _, i1, _ = _colmax_argmax(x2)
    pen = jnp.where(i0 == i1, jnp.float32(-1e16), jnp.float32(0.0))
    out_ref[...] = jnp.maximum(v0 + v0 + pen, v0 + v1)[None, :]


def _tc_full(inputs):
    grid = TC_COLS // TC_BLK
    return pl.pallas_call(
        _tc_full_body,
        grid=(grid,),
        in_specs=[
            pl.BlockSpec((ROWS, TC_BLK), lambda j: (0, SC_BLKS + j)),
            pl.BlockSpec((ROWS, TC_BLK),
                         lambda j: (0, HALF // TC_BLK + SC_BLKS + j)),
        ],
        out_specs=pl.BlockSpec((1, TC_BLK), lambda j: (0, j)),
        out_shape=jax.ShapeDtypeStruct((1, TC_COLS), jnp.float32),
    )(inputs, inputs)


@jax.jit
def _run(inputs):
    out_sc = _sc_part(inputs)
    out_tc = _tc_full(inputs)
    return jnp.concatenate([out_sc.reshape(1, SC_COLS), out_tc], axis=1)


def kernel(inputs):
    return _run(inputs)
